# Initial kernel scaffold; baseline (speedup 1.0000x reference)
import jax
import jax.numpy as jnp
from jax.experimental import pallas as pl


def kernel(x, adj, W, b, att_w, att_b):
    raise NotImplementedError("write your pallas kernel here")


# speedup vs baseline: 98.6031x; 98.6031x over previous
import dataclasses
import functools

import jax
import jax.numpy as jnp
from jax import lax
from jax.experimental import pallas as pl
from jax.experimental.pallas import tpu as pltpu
from jax.experimental.pallas import tpu_sc as plsc

N = 4096
D = 128
BLK = 256
NBLK = N // BLK
HIGHEST = lax.Precision.HIGHEST


def _prep_body(x_ref, w_ref, b_ref, wl_ref, wr_ref, xl_ref, esl_ref, esr_ref):
    xl = lax.dot_general(x_ref[...], w_ref[...], (((1,), (1,)), ((), ())),
                         precision=HIGHEST) + b_ref[...]
    xl_ref[...] = xl
    sl = lax.dot_general(wl_ref[...], xl, (((1,), (1,)), ((), ())),
                         precision=HIGHEST)
    sr = lax.dot_general(wr_ref[...], xl, (((1,), (1,)), ((), ())),
                         precision=HIGHEST)
    esl_ref[...] = jnp.exp(sl - jnp.max(sl))
    esr_ref[...] = jnp.exp(sr - jnp.max(sr))


def _prep(x, W, b2, wl, wr, interpret=False):
    return pl.pallas_call(
        _prep_body,
        out_shape=(
            jax.ShapeDtypeStruct((N, D), jnp.float32),
            jax.ShapeDtypeStruct((1, N), jnp.float32),
            jax.ShapeDtypeStruct((1, N), jnp.float32),
        ),
        interpret=interpret,
    )(x, W, b2, wl, wr)


def _passa_body(adj_ref, esr_ref, esl_ref, offs_ref, cnts_ref, denom_ref,
                carry_ref):
    pid = pl.program_id(0)

    @pl.when(pid == 0)
    def _():
        carry_ref[0] = 0
        denom_ref[...] = jnp.zeros((1, 1), jnp.float32)

    mask = (adj_ref[...] == 1).astype(jnp.float32)
    rowsum = lax.dot_general(esr_ref[...], mask, (((1,), (1,)), ((), ())),
                             precision=HIGHEST)
    cnts = lax.dot_general(jnp.ones((1, N), jnp.float32), mask,
                           (((1,), (1,)), ((), ())),
                           precision=HIGHEST)
    row_i = lax.broadcasted_iota(jnp.int32, (BLK, BLK), 0)
    col_i = lax.broadcasted_iota(jnp.int32, (BLK, BLK), 1)
    tri = (row_i < col_i).astype(jnp.float32)
    excl = lax.dot_general(cnts, tri, (((1,), (0,)), ((), ())),
                           precision=HIGHEST)
    c0 = carry_ref[0]
    offs_ref[...] = (excl + c0.astype(jnp.float32)).astype(
        jnp.int32).reshape(1, 1, BLK)
    cnts_ref[...] = cnts.astype(jnp.int32).reshape(1, 1, BLK)
    denom_ref[...] = denom_ref[...] + jnp.sum(esl_ref[...] * rowsum)
    carry_ref[0] = c0 + jnp.sum(cnts).astype(jnp.int32)


def _passa(adj, esr, esl, interpret=False):
    return pl.pallas_call(
        _passa_body,
        grid=(NBLK,),
        in_specs=[
            pl.BlockSpec((BLK, N), lambda i: (i, 0)),
            pl.BlockSpec((1, N), lambda i: (0, 0)),
            pl.BlockSpec((1, BLK), lambda i: (0, i)),
        ],
        out_specs=(
            pl.BlockSpec((1, 1, BLK), lambda i: (i, 0, 0)),
            pl.BlockSpec((1, 1, BLK), lambda i: (i, 0, 0)),
            pl.BlockSpec((1, 1), lambda i: (0, 0)),
        ),
        out_shape=(
            jax.ShapeDtypeStruct((NBLK, 1, BLK), jnp.int32),
            jax.ShapeDtypeStruct((NBLK, 1, BLK), jnp.int32),
            jax.ShapeDtypeStruct((1, 1), jnp.float32),
        ),
        scratch_shapes=[pltpu.SMEM((1,), jnp.int32)],
        interpret=interpret,
    )(adj, esr, esl)


def _extract_sc(adj, offs, cnts, esl1, esr1):
    mesh = plsc.VectorSubcoreMesh(core_axis_name="c", subcore_axis_name="s")
    cp = pltpu.CompilerParams()
    if "needs_layout_passes" in pltpu.CompilerParams.__dataclass_fields__:
        cp = dataclasses.replace(cp, needs_layout_passes=False)

    @functools.partial(
        pl.kernel,
        out_type=jax.ShapeDtypeStruct((32, N), jnp.float32),
        mesh=mesh,
        compiler_params=cp,
        scratch_types=[
            pltpu.VMEM((N,), jnp.int32),
            pltpu.VMEM((N,), jnp.int32),
            pltpu.VMEM((N,), jnp.int32),
            pltpu.VMEM((N,), jnp.float32),
            pltpu.VMEM((N,), jnp.float32),
            pltpu.VMEM((N,), jnp.float32),
        ],
    )
    def k(adj_hbm, offs_hbm, cnts_hbm, esl_hbm, esr_hbm, out_hbm,
          row_v, offs_v, cnts_v, esl_v, esr_v, acc_v):
        wid = lax.axis_index("s") * 2 + lax.axis_index("c")
        pltpu.sync_copy(offs_hbm, offs_v)
        pltpu.sync_copy(cnts_hbm, cnts_v)
        pltpu.sync_copy(esl_hbm, esl_v)
        pltpu.sync_copy(esr_hbm, esr_v)

        @pl.loop(0, N, step=16)
        def _(c):
            acc_v[pl.ds(c, 16)] = jnp.zeros((16,), jnp.float32)

        @pl.loop(0, N // 32)
        def _(t):
            i = t * 32 + wid
            iv = jnp.full((16,), i, jnp.int32)
            off = jnp.max(plsc.load_gather(offs_v, [iv]))
            cnt = jnp.max(plsc.load_gather(cnts_v, [iv]))

            @pl.when((off < N) & (cnt > 0))
            def _():
                pltpu.sync_copy(adj_hbm.at[i], row_v)
                eslb = plsc.load_gather(esl_v, [iv])

                def chunk(c, tot):
                    v = row_v[pl.ds(c * 16, 16)]
                    m = v == 1
                    ones = jnp.where(m, 1, 0).astype(jnp.int32)
                    cs = plsc.cumsum(ones)
                    rank = off + tot + cs - 1
                    valid = m & (rank < N)
                    rank_safe = jnp.where(valid, rank, 0)
                    val = eslb * esr_v[pl.ds(c * 16, 16)]
                    plsc.store_scatter(acc_v, [rank_safe], val, mask=valid)
                    return tot + jnp.sum(ones)

                lax.fori_loop(0, N // 16, chunk, jnp.int32(0))

        pltpu.sync_copy(acc_v, out_hbm.at[wid])

    return k(adj, offs, cnts, esl1, esr1)


def _passb_body(adj_ref, aparts_ref, xl_ref, denom_ref, out_ref):
    mask = (adj_ref[...] == 1).astype(jnp.float32)
    a = jnp.sum(aparts_ref[...], axis=0, keepdims=True)
    d = denom_ref[...]
    d = jnp.where(d > 0.0, d, 1.0)
    mw = mask * a
    ob = lax.dot_general(mw, xl_ref[...], (((1,), (0,)), ((), ())),
                         precision=HIGHEST)
    out_ref[...] = jnp.maximum(ob / d, 0.0)


def _passb(adj, aparts, xl, denom, interpret=False):
    return pl.pallas_call(
        _passb_body,
        grid=(NBLK,),
        in_specs=[
            pl.BlockSpec((BLK, N), lambda i: (i, 0)),
            pl.BlockSpec((32, N), lambda i: (0, 0)),
            pl.BlockSpec((N, D), lambda i: (0, 0)),
            pl.BlockSpec((1, 1), lambda i: (0, 0)),
        ],
        out_specs=pl.BlockSpec((BLK, D), lambda i: (i, 0)),
        out_shape=jax.ShapeDtypeStruct((N, D), jnp.float32),
        interpret=interpret,
    )(adj, aparts, xl, denom)


def kernel(x, adj, W, b, att_w, att_b):
    wl = att_w[:, :D]
    wr = att_w[:, D:]
    b2 = b.reshape(1, D)
    xl, esl, esr = _prep(x, W, b2, wl, wr)
    offs3, cnts3, denom = _passa(adj, esr, esl)
    offs = offs3.reshape(N)
    cnts = cnts3.reshape(N)
    aparts = _extract_sc(adj, offs, cnts, esl.reshape(N), esr.reshape(N))
    return _passb(adj, aparts, xl, denom)


# speedup vs baseline: 214.2231x; 2.1726x over previous
import dataclasses
import functools

import jax
import jax.numpy as jnp
from jax import lax
from jax.experimental import pallas as pl
from jax.experimental.pallas import tpu as pltpu
from jax.experimental.pallas import tpu_sc as plsc

N = 4096
D = 128
BLK = 256
NBLK = N // BLK
HIGHEST = lax.Precision.HIGHEST


def _prep_body(x_ref, w_ref, b_ref, wl_ref, wr_ref, xl_ref, esl_ref, esr_ref):
    xl = lax.dot_general(x_ref[...], w_ref[...], (((1,), (1,)), ((), ())),
                         precision=HIGHEST) + b_ref[...]
    xl_ref[...] = xl
    sl = lax.dot_general(wl_ref[...], xl, (((1,), (1,)), ((), ())),
                         precision=HIGHEST)
    sr = lax.dot_general(wr_ref[...], xl, (((1,), (1,)), ((), ())),
                         precision=HIGHEST)
    esl_ref[...] = jnp.exp(sl - jnp.max(sl))
    esr_ref[...] = jnp.exp(sr - jnp.max(sr))


def _prep(x, W, b2, wl, wr, interpret=False):
    return pl.pallas_call(
        _prep_body,
        out_shape=(
            jax.ShapeDtypeStruct((N, D), jnp.float32),
            jax.ShapeDtypeStruct((1, N), jnp.float32),
            jax.ShapeDtypeStruct((1, N), jnp.float32),
        ),
        interpret=interpret,
    )(x, W, b2, wl, wr)


def _passa_body(adj_ref, esr_ref, esl_ref, offs_ref, cnts_ref, denom_ref,
                carry_ref):
    pid = pl.program_id(0)

    @pl.when(pid == 0)
    def _():
        carry_ref[0] = 0
        denom_ref[...] = jnp.zeros((1, 1), jnp.float32)

    mask = (adj_ref[...] == 1).astype(jnp.float32)
    lhs = jnp.concatenate([esr_ref[...], jnp.ones((1, N), jnp.float32)], 0)
    rc = lax.dot_general(lhs, mask, (((1,), (1,)), ((), ())))
    rowsum = rc[0:1, :]
    cnts = rc[1:2, :]
    row_i = lax.broadcasted_iota(jnp.int32, (BLK, BLK), 0)
    col_i = lax.broadcasted_iota(jnp.int32, (BLK, BLK), 1)
    tri = (row_i < col_i).astype(jnp.float32)
    excl = lax.dot_general(cnts, tri, (((1,), (0,)), ((), ())),
                           precision=HIGHEST)
    c0 = carry_ref[0]
    offs_ref[...] = (excl + c0.astype(jnp.float32)).astype(
        jnp.int32).reshape(1, 1, BLK)
    cnts_ref[...] = cnts.astype(jnp.int32).reshape(1, 1, BLK)
    denom_ref[...] = denom_ref[...] + jnp.sum(esl_ref[...] * rowsum)
    carry_ref[0] = c0 + jnp.sum(cnts).astype(jnp.int32)


def _passa(adj, esr, esl, interpret=False):
    return pl.pallas_call(
        _passa_body,
        grid=(NBLK,),
        in_specs=[
            pl.BlockSpec((BLK, N), lambda i: (i, 0)),
            pl.BlockSpec((1, N), lambda i: (0, 0)),
            pl.BlockSpec((1, BLK), lambda i: (0, i)),
        ],
        out_specs=(
            pl.BlockSpec((1, 1, BLK), lambda i: (i, 0, 0)),
            pl.BlockSpec((1, 1, BLK), lambda i: (i, 0, 0)),
            pl.BlockSpec((1, 1), lambda i: (0, 0)),
        ),
        out_shape=(
            jax.ShapeDtypeStruct((NBLK, 1, BLK), jnp.int32),
            jax.ShapeDtypeStruct((NBLK, 1, BLK), jnp.int32),
            jax.ShapeDtypeStruct((1, 1), jnp.float32),
        ),
        scratch_shapes=[pltpu.SMEM((1,), jnp.int32)],
        interpret=interpret,
    )(adj, esr, esl)


def _extract_sc(adj, offs, cnts, esl1, esr1):
    mesh = plsc.VectorSubcoreMesh(core_axis_name="c", subcore_axis_name="s")
    cp = pltpu.CompilerParams()
    if "needs_layout_passes" in pltpu.CompilerParams.__dataclass_fields__:
        cp = dataclasses.replace(cp, needs_layout_passes=False)

    @functools.partial(
        pl.kernel,
        out_type=jax.ShapeDtypeStruct((32, N), jnp.float32),
        mesh=mesh,
        compiler_params=cp,
        scratch_types=[
            pltpu.VMEM((N,), jnp.int32),
            pltpu.VMEM((N,), jnp.int32),
            pltpu.VMEM((N,), jnp.int32),
            pltpu.VMEM((N,), jnp.float32),
            pltpu.VMEM((N,), jnp.float32),
            pltpu.VMEM((N,), jnp.float32),
        ],
    )
    def k(adj_hbm, offs_hbm, cnts_hbm, esl_hbm, esr_hbm, out_hbm,
          row_v, offs_v, cnts_v, esl_v, esr_v, acc_v):
        wid = lax.axis_index("s") * 2 + lax.axis_index("c")
        pltpu.sync_copy(offs_hbm, offs_v)
        pltpu.sync_copy(cnts_hbm, cnts_v)
        pltpu.sync_copy(esl_hbm, esl_v)
        pltpu.sync_copy(esr_hbm, esr_v)

        @pl.loop(0, N, step=16)
        def _(c):
            acc_v[pl.ds(c, 16)] = jnp.zeros((16,), jnp.float32)

        @pl.loop(0, N // 32)
        def _(t):
            i = t * 32 + wid
            iv = jnp.full((16,), i, jnp.int32)
            off = jnp.max(plsc.load_gather(offs_v, [iv]))
            cnt = jnp.max(plsc.load_gather(cnts_v, [iv]))

            @pl.when((off < N) & (cnt > 0))
            def _():
                pltpu.sync_copy(adj_hbm.at[i], row_v)
                eslb = plsc.load_gather(esl_v, [iv])

                def chunk(c, tot):
                    v = row_v[pl.ds(c * 16, 16)]
                    m = v == 1
                    ones = jnp.where(m, 1, 0).astype(jnp.int32)
                    cs = plsc.cumsum(ones)
                    rank = off + tot + cs - 1
                    valid = m & (rank < N)
                    rank_safe = jnp.where(valid, rank, 0)
                    val = eslb * esr_v[pl.ds(c * 16, 16)]
                    plsc.store_scatter(acc_v, [rank_safe], val, mask=valid)
                    return tot + jnp.sum(ones)

                lax.fori_loop(0, N // 16, chunk, jnp.int32(0))

        pltpu.sync_copy(acc_v, out_hbm.at[wid])

    return k(adj, offs, cnts, esl1, esr1)


def _passb_body(adj_ref, aparts_ref, xl_ref, denom_ref, out_ref):
    mask = (adj_ref[...] == 1).astype(jnp.float32)
    a = jnp.sum(aparts_ref[...], axis=0, keepdims=True)
    d = denom_ref[...]
    d = jnp.where(d > 0.0, d, 1.0)
    mw = mask * (a / d)
    ob = lax.dot_general(mw, xl_ref[...], (((1,), (0,)), ((), ())))
    out_ref[...] = jnp.maximum(ob, 0.0)


def _passb(adj, aparts, xl, denom, interpret=False):
    return pl.pallas_call(
        _passb_body,
        grid=(NBLK,),
        in_specs=[
            pl.BlockSpec((BLK, N), lambda i: (i, 0)),
            pl.BlockSpec((32, N), lambda i: (0, 0)),
            pl.BlockSpec((N, D), lambda i: (0, 0)),
            pl.BlockSpec((1, 1), lambda i: (0, 0)),
        ],
        out_specs=pl.BlockSpec((BLK, D), lambda i: (i, 0)),
        out_shape=jax.ShapeDtypeStruct((N, D), jnp.float32),
        interpret=interpret,
    )(adj, aparts, xl, denom)


def kernel(x, adj, W, b, att_w, att_b):
    wl = att_w[:, :D]
    wr = att_w[:, D:]
    b2 = b.reshape(1, D)
    xl, esl, esr = _prep(x, W, b2, wl, wr)
    offs3, cnts3, denom = _passa(adj, esr, esl)
    offs = offs3.reshape(N)
    cnts = cnts3.reshape(N)
    aparts = _extract_sc(adj, offs, cnts, esl.reshape(N), esr.reshape(N))
    return _passb(adj, aparts, xl, denom)


# speedup vs baseline: 214.9737x; 1.0035x over previous
import dataclasses
import functools

import jax
import jax.numpy as jnp
from jax import lax
from jax.experimental import pallas as pl
from jax.experimental.pallas import tpu as pltpu
from jax.experimental.pallas import tpu_sc as plsc

N = 4096
D = 128
BLK = 256
NBLK = N // BLK
HIGHEST = lax.Precision.HIGHEST


def _prep_body(x_ref, w_ref, b_ref, wl_ref, wr_ref, xl_ref, esl_ref, esr_ref):
    xl = lax.dot_general(x_ref[...], w_ref[...], (((1,), (1,)), ((), ())),
                         precision=HIGHEST) + b_ref[...]
    xl_ref[...] = xl
    sl = lax.dot_general(wl_ref[...], xl, (((1,), (1,)), ((), ())),
                         precision=HIGHEST)
    sr = lax.dot_general(wr_ref[...], xl, (((1,), (1,)), ((), ())),
                         precision=HIGHEST)
    esl_ref[...] = jnp.exp(sl - jnp.max(sl))
    esr_ref[...] = jnp.exp(sr - jnp.max(sr))


def _prep(x, W, b2, wl, wr, interpret=False):
    return pl.pallas_call(
        _prep_body,
        out_shape=(
            jax.ShapeDtypeStruct((N, D), jnp.float32),
            jax.ShapeDtypeStruct((1, N), jnp.float32),
            jax.ShapeDtypeStruct((1, N), jnp.float32),
        ),
        interpret=interpret,
    )(x, W, b2, wl, wr)


def _passa_body(adj_ref, esr_ref, esl_ref, offs_ref, cnts_ref, denom_ref,
                rowlim_ref, carry_ref):
    pid = pl.program_id(0)

    @pl.when(pid == 0)
    def _():
        carry_ref[0] = 0
        denom_ref[...] = jnp.zeros((1, 1), jnp.float32)
        rowlim_ref[...] = jnp.zeros((1, 1), jnp.int32)

    mask = (adj_ref[...] == 1).astype(jnp.float32)
    lhs = jnp.concatenate([esr_ref[...], jnp.ones((1, N), jnp.float32)], 0)
    rc = lax.dot_general(lhs, mask, (((1,), (1,)), ((), ())))
    rowsum = rc[0:1, :]
    cnts = rc[1:2, :]
    row_i = lax.broadcasted_iota(jnp.int32, (BLK, BLK), 0)
    col_i = lax.broadcasted_iota(jnp.int32, (BLK, BLK), 1)
    tri = (row_i < col_i).astype(jnp.float32)
    excl = lax.dot_general(cnts, tri, (((1,), (0,)), ((), ())),
                           precision=HIGHEST)
    c0 = carry_ref[0]
    offs_i = (excl + c0.astype(jnp.float32)).astype(jnp.int32)
    offs_ref[...] = offs_i.reshape(1, 1, BLK)
    cnts_ref[...] = cnts.astype(jnp.int32).reshape(1, 1, BLK)
    denom_ref[...] = denom_ref[...] + jnp.sum(esl_ref[...] * rowsum)
    rowlim_ref[...] = rowlim_ref[...] + jnp.sum(
        (offs_i < N).astype(jnp.int32)).reshape(1, 1)
    carry_ref[0] = c0 + jnp.sum(cnts).astype(jnp.int32)


def _passa(adj, esr, esl, interpret=False):
    return pl.pallas_call(
        _passa_body,
        grid=(NBLK,),
        in_specs=[
            pl.BlockSpec((BLK, N), lambda i: (i, 0)),
            pl.BlockSpec((1, N), lambda i: (0, 0)),
            pl.BlockSpec((1, BLK), lambda i: (0, i)),
        ],
        out_specs=(
            pl.BlockSpec((1, 1, BLK), lambda i: (i, 0, 0)),
            pl.BlockSpec((1, 1, BLK), lambda i: (i, 0, 0)),
            pl.BlockSpec((1, 1), lambda i: (0, 0)),
            pl.BlockSpec((1, 1), lambda i: (0, 0)),
        ),
        out_shape=(
            jax.ShapeDtypeStruct((NBLK, 1, BLK), jnp.int32),
            jax.ShapeDtypeStruct((NBLK, 1, BLK), jnp.int32),
            jax.ShapeDtypeStruct((1, 1), jnp.float32),
            jax.ShapeDtypeStruct((1, 1), jnp.int32),
        ),
        scratch_shapes=[pltpu.SMEM((1,), jnp.int32)],
        interpret=interpret,
    )(adj, esr, esl)


def _extract_sc(adj, offs, cnts, rowlim, esl1, esr1):
    mesh = plsc.VectorSubcoreMesh(core_axis_name="c", subcore_axis_name="s")
    cp = pltpu.CompilerParams()
    if "needs_layout_passes" in pltpu.CompilerParams.__dataclass_fields__:
        cp = dataclasses.replace(cp, needs_layout_passes=False)

    @functools.partial(
        pl.kernel,
        out_type=jax.ShapeDtypeStruct((32, N), jnp.float32),
        mesh=mesh,
        compiler_params=cp,
        scratch_types=[
            pltpu.VMEM((N,), jnp.int32),
            pltpu.VMEM((N,), jnp.int32),
            pltpu.VMEM((N,), jnp.int32),
            pltpu.VMEM((16,), jnp.int32),
            pltpu.VMEM((N,), jnp.float32),
            pltpu.VMEM((N,), jnp.float32),
            pltpu.VMEM((N,), jnp.float32),
        ],
    )
    def k(adj_hbm, offs_hbm, cnts_hbm, rowlim_hbm, esl_hbm, esr_hbm, out_hbm,
          row_v, offs_v, cnts_v, rl_v, esl_v, esr_v, acc_v):
        wid = lax.axis_index("s") * 2 + lax.axis_index("c")
        pltpu.sync_copy(offs_hbm, offs_v)
        pltpu.sync_copy(cnts_hbm, cnts_v)
        pltpu.sync_copy(rowlim_hbm, rl_v)
        pltpu.sync_copy(esl_hbm, esl_v)
        pltpu.sync_copy(esr_hbm, esr_v)

        @pl.loop(0, N, step=16)
        def _(c):
            acc_v[pl.ds(c, 16)] = jnp.zeros((16,), jnp.float32)

        rl = jnp.max(plsc.load_gather(rl_v, [jnp.zeros((16,), jnp.int32)]))
        nt = jnp.maximum(rl - wid + 31, 0) // 32

        def trip(t, _):
            i = t * 32 + wid
            iv = jnp.full((16,), i, jnp.int32)
            off = jnp.max(plsc.load_gather(offs_v, [iv]))
            cnt = jnp.max(plsc.load_gather(cnts_v, [iv]))

            @pl.when((off < N) & (cnt > 0))
            def _():
                pltpu.sync_copy(adj_hbm.at[i], row_v)
                eslb = plsc.load_gather(esl_v, [iv])

                def chunk(c, tot):
                    v = row_v[pl.ds(c * 16, 16)]
                    m = v == 1
                    ones = jnp.where(m, 1, 0).astype(jnp.int32)
                    cs = plsc.cumsum(ones)
                    rank = off + tot + cs - 1
                    valid = m & (rank < N)
                    rank_safe = jnp.where(valid, rank, 0)
                    val = eslb * esr_v[pl.ds(c * 16, 16)]
                    plsc.store_scatter(acc_v, [rank_safe], val, mask=valid)
                    return tot + jnp.sum(ones)

                lax.fori_loop(0, N // 16, chunk, jnp.int32(0))

            return 0

        lax.fori_loop(0, nt, trip, jnp.int32(0))

        pltpu.sync_copy(acc_v, out_hbm.at[wid])

    return k(adj, offs, cnts, rowlim, esl1, esr1)


def _passb_body(adj_ref, aparts_ref, xl_ref, denom_ref, out_ref):
    mask = (adj_ref[...] == 1).astype(jnp.float32)
    a = jnp.sum(aparts_ref[...], axis=0, keepdims=True)
    d = denom_ref[...]
    d = jnp.where(d > 0.0, d, 1.0)
    mw = mask * (a / d)
    ob = lax.dot_general(mw, xl_ref[...], (((1,), (0,)), ((), ())))
    out_ref[...] = jnp.maximum(ob, 0.0)


def _passb(adj, aparts, xl, denom, interpret=False):
    return pl.pallas_call(
        _passb_body,
        grid=(NBLK,),
        in_specs=[
            pl.BlockSpec((BLK, N), lambda i: (i, 0)),
            pl.BlockSpec((32, N), lambda i: (0, 0)),
            pl.BlockSpec((N, D), lambda i: (0, 0)),
            pl.BlockSpec((1, 1), lambda i: (0, 0)),
        ],
        out_specs=pl.BlockSpec((BLK, D), lambda i: (i, 0)),
        out_shape=jax.ShapeDtypeStruct((N, D), jnp.float32),
        interpret=interpret,
    )(adj, aparts, xl, denom)


def kernel(x, adj, W, b, att_w, att_b):
    wl = att_w[:, :D]
    wr = att_w[:, D:]
    b2 = b.reshape(1, D)
    xl, esl, esr = _prep(x, W, b2, wl, wr)
    offs3, cnts3, denom, rowlim = _passa(adj, esr, esl)
    offs = offs3.reshape(N)
    cnts = cnts3.reshape(N)
    rl16 = jnp.broadcast_to(rowlim.reshape(1), (16,))
    aparts = _extract_sc(adj, offs, cnts, rl16, esl.reshape(N),
                         esr.reshape(N))
    return _passb(adj, aparts, xl, denom)


# speedup vs baseline: 224.7579x; 1.0455x over previous
import dataclasses
import functools

import jax
import jax.numpy as jnp
from jax import lax
from jax.experimental import pallas as pl
from jax.experimental.pallas import tpu as pltpu
from jax.experimental.pallas import tpu_sc as plsc

N = 4096
D = 128
BLK = 256
NBLK = N // BLK
HIGHEST = lax.Precision.HIGHEST


def _prep_body(x_ref, w_ref, b_ref, wl_ref, wr_ref, xl_ref, esl_ref, esr_ref):
    xl = lax.dot_general(x_ref[...], w_ref[...], (((1,), (1,)), ((), ())),
                         precision=HIGHEST) + b_ref[...]
    xl_ref[...] = xl
    sl = lax.dot_general(wl_ref[...], xl, (((1,), (1,)), ((), ())),
                         precision=HIGHEST)
    sr = lax.dot_general(wr_ref[...], xl, (((1,), (1,)), ((), ())),
                         precision=HIGHEST)
    esl_ref[...] = jnp.exp(sl - jnp.max(sl))
    esr_ref[...] = jnp.exp(sr - jnp.max(sr))


def _prep(x, W, b2, wl, wr, interpret=False):
    return pl.pallas_call(
        _prep_body,
        out_shape=(
            jax.ShapeDtypeStruct((N, D), jnp.float32),
            jax.ShapeDtypeStruct((1, N), jnp.float32),
            jax.ShapeDtypeStruct((1, N), jnp.float32),
        ),
        interpret=interpret,
    )(x, W, b2, wl, wr)


def _passa_body(adj_ref, esr_ref, esl_ref, offs_ref, cnts_ref, denom_ref,
                rowlim_ref, mask8_ref, carry_ref):
    pid = pl.program_id(0)

    @pl.when(pid == 0)
    def _():
        carry_ref[0] = 0
        denom_ref[...] = jnp.zeros((1, 1), jnp.float32)
        rowlim_ref[...] = jnp.zeros((1, 1), jnp.int32)

    mask = (adj_ref[...] == 1).astype(jnp.float32)
    mask8_ref[...] = mask.astype(jnp.int8)
    lhs = jnp.concatenate([esr_ref[...], jnp.ones((1, N), jnp.float32)], 0)
    rc = lax.dot_general(lhs, mask, (((1,), (1,)), ((), ())))
    rowsum = rc[0:1, :]
    cnts = rc[1:2, :]
    row_i = lax.broadcasted_iota(jnp.int32, (BLK, BLK), 0)
    col_i = lax.broadcasted_iota(jnp.int32, (BLK, BLK), 1)
    tri = (row_i < col_i).astype(jnp.float32)
    excl = lax.dot_general(cnts, tri, (((1,), (0,)), ((), ())),
                           precision=HIGHEST)
    c0 = carry_ref[0]
    offs_i = (excl + c0.astype(jnp.float32)).astype(jnp.int32)
    offs_ref[...] = offs_i.reshape(1, 1, BLK)
    cnts_ref[...] = cnts.astype(jnp.int32).reshape(1, 1, BLK)
    denom_ref[...] = denom_ref[...] + jnp.sum(esl_ref[...] * rowsum)
    rowlim_ref[...] = rowlim_ref[...] + jnp.sum(
        (offs_i < N).astype(jnp.int32)).reshape(1, 1)
    carry_ref[0] = c0 + jnp.sum(cnts).astype(jnp.int32)


def _passa(adj, esr, esl, interpret=False):
    return pl.pallas_call(
        _passa_body,
        grid=(NBLK,),
        in_specs=[
            pl.BlockSpec((BLK, N), lambda i: (i, 0)),
            pl.BlockSpec((1, N), lambda i: (0, 0)),
            pl.BlockSpec((1, BLK), lambda i: (0, i)),
        ],
        out_specs=(
            pl.BlockSpec((1, 1, BLK), lambda i: (i, 0, 0)),
            pl.BlockSpec((1, 1, BLK), lambda i: (i, 0, 0)),
            pl.BlockSpec((1, 1), lambda i: (0, 0)),
            pl.BlockSpec((1, 1), lambda i: (0, 0)),
            pl.BlockSpec((BLK, N), lambda i: (i, 0)),
        ),
        out_shape=(
            jax.ShapeDtypeStruct((NBLK, 1, BLK), jnp.int32),
            jax.ShapeDtypeStruct((NBLK, 1, BLK), jnp.int32),
            jax.ShapeDtypeStruct((1, 1), jnp.float32),
            jax.ShapeDtypeStruct((1, 1), jnp.int32),
            jax.ShapeDtypeStruct((N, N), jnp.int8),
        ),
        scratch_shapes=[pltpu.SMEM((1,), jnp.int32)],
        interpret=interpret,
    )(adj, esr, esl)


def _extract_sc(adj, offs, cnts, rowlim, esl1, esr1):
    mesh = plsc.VectorSubcoreMesh(core_axis_name="c", subcore_axis_name="s")
    cp = pltpu.CompilerParams()
    if "needs_layout_passes" in pltpu.CompilerParams.__dataclass_fields__:
        cp = dataclasses.replace(cp, needs_layout_passes=False)

    @functools.partial(
        pl.kernel,
        out_type=jax.ShapeDtypeStruct((32, N), jnp.float32),
        mesh=mesh,
        compiler_params=cp,
        scratch_types=[
            pltpu.VMEM((N,), jnp.int32),
            pltpu.VMEM((N,), jnp.int32),
            pltpu.VMEM((N,), jnp.int32),
            pltpu.VMEM((16,), jnp.int32),
            pltpu.VMEM((N,), jnp.float32),
            pltpu.VMEM((N,), jnp.float32),
            pltpu.VMEM((N,), jnp.float32),
        ],
    )
    def k(adj_hbm, offs_hbm, cnts_hbm, rowlim_hbm, esl_hbm, esr_hbm, out_hbm,
          row_v, offs_v, cnts_v, rl_v, esl_v, esr_v, acc_v):
        wid = lax.axis_index("s") * 2 + lax.axis_index("c")
        pltpu.sync_copy(offs_hbm, offs_v)
        pltpu.sync_copy(cnts_hbm, cnts_v)
        pltpu.sync_copy(rowlim_hbm, rl_v)
        pltpu.sync_copy(esl_hbm, esl_v)
        pltpu.sync_copy(esr_hbm, esr_v)

        @pl.loop(0, N, step=16)
        def _(c):
            acc_v[pl.ds(c, 16)] = jnp.zeros((16,), jnp.float32)

        rl = jnp.max(plsc.load_gather(rl_v, [jnp.zeros((16,), jnp.int32)]))
        nt = jnp.maximum(rl - wid + 31, 0) // 32

        def trip(t, _):
            i = t * 32 + wid
            iv = jnp.full((16,), i, jnp.int32)
            off = jnp.max(plsc.load_gather(offs_v, [iv]))
            cnt = jnp.max(plsc.load_gather(cnts_v, [iv]))

            @pl.when((off < N) & (cnt > 0))
            def _():
                pltpu.sync_copy(adj_hbm.at[i], row_v)
                eslb = plsc.load_gather(esl_v, [iv])

                def chunk(c, tot):
                    v = row_v[pl.ds(c * 16, 16)]
                    m = v == 1
                    ones = jnp.where(m, 1, 0).astype(jnp.int32)
                    cs = plsc.cumsum(ones)
                    rank = off + tot + cs - 1
                    valid = m & (rank < N)
                    rank_safe = jnp.where(valid, rank, 0)
                    val = eslb * esr_v[pl.ds(c * 16, 16)]
                    plsc.store_scatter(acc_v, [rank_safe], val, mask=valid)
                    return tot + jnp.sum(ones)

                lax.fori_loop(0, N // 16, chunk, jnp.int32(0))

            return 0

        lax.fori_loop(0, nt, trip, jnp.int32(0))

        pltpu.sync_copy(acc_v, out_hbm.at[wid])

    return k(adj, offs, cnts, rowlim, esl1, esr1)


def _passb_body(mask8_ref, aparts_ref, xl_ref, denom_ref, out_ref):
    mask = mask8_ref[...].astype(jnp.float32)
    a = jnp.sum(aparts_ref[...], axis=0, keepdims=True)
    d = denom_ref[...]
    d = jnp.where(d > 0.0, d, 1.0)
    mw = mask * (a / d)
    ob = lax.dot_general(mw, xl_ref[...], (((1,), (0,)), ((), ())))
    out_ref[...] = jnp.maximum(ob, 0.0)


def _passb(mask8, aparts, xl, denom, interpret=False):
    return pl.pallas_call(
        _passb_body,
        grid=(NBLK,),
        in_specs=[
            pl.BlockSpec((BLK, N), lambda i: (i, 0)),
            pl.BlockSpec((32, N), lambda i: (0, 0)),
            pl.BlockSpec((N, D), lambda i: (0, 0)),
            pl.BlockSpec((1, 1), lambda i: (0, 0)),
        ],
        out_specs=pl.BlockSpec((BLK, D), lambda i: (i, 0)),
        out_shape=jax.ShapeDtypeStruct((N, D), jnp.float32),
        interpret=interpret,
    )(mask8, aparts, xl, denom)


def kernel(x, adj, W, b, att_w, att_b):
    wl = att_w[:, :D]
    wr = att_w[:, D:]
    b2 = b.reshape(1, D)
    xl, esl, esr = _prep(x, W, b2, wl, wr)
    offs3, cnts3, denom, rowlim, mask8 = _passa(adj, esr, esl)
    offs = offs3.reshape(N)
    cnts = cnts3.reshape(N)
    rl16 = jnp.broadcast_to(rowlim.reshape(1), (16,))
    aparts = _extract_sc(adj, offs, cnts, rl16, esl.reshape(N),
                         esr.reshape(N))
    return _passb(mask8, aparts, xl, denom)


# speedup vs baseline: 232.6786x; 1.0352x over previous
import dataclasses
import functools

import jax
import jax.numpy as jnp
from jax import lax
from jax.experimental import pallas as pl
from jax.experimental.pallas import tpu as pltpu
from jax.experimental.pallas import tpu_sc as plsc

N = 4096
D = 128
BLK = 256
NBLK = N // BLK
HIGHEST = lax.Precision.HIGHEST


def _passa_body(adj_ref, x_ref, w_ref, b_ref, wl_ref, wr_ref,
                offs_ref, cnts_ref, denom_ref, rowlim_ref, mask8_ref,
                xl_ref, esl_ref, esr_ref,
                carry_ref):
    pid = pl.program_id(0)

    @pl.when(pid == 0)
    def _():
        carry_ref[0] = 0
        denom_ref[...] = jnp.zeros((1, 1), jnp.float32)
        rowlim_ref[...] = jnp.zeros((1, 1), jnp.int32)
        xl = lax.dot_general(x_ref[...], w_ref[...], (((1,), (1,)), ((), ())),
                             precision=HIGHEST) + b_ref[...]
        xl_ref[...] = xl
        sl = lax.dot_general(wl_ref[...], xl, (((1,), (1,)), ((), ())),
                             precision=HIGHEST)
        sr = lax.dot_general(wr_ref[...], xl, (((1,), (1,)), ((), ())),
                             precision=HIGHEST)
        esl_ref[...] = jnp.exp(sl - jnp.max(sl))
        esr_ref[...] = jnp.exp(sr - jnp.max(sr))

    mask = (adj_ref[...] == 1).astype(jnp.float32)
    mask8_ref[...] = mask.astype(jnp.int8)
    lhs = jnp.concatenate([esr_ref[...], jnp.ones((1, N), jnp.float32)], 0)
    rc = lax.dot_general(lhs, mask, (((1,), (1,)), ((), ())))
    rowsum = rc[0:1, :]
    cnts = rc[1:2, :]
    row_i = lax.broadcasted_iota(jnp.int32, (BLK, BLK), 0)
    col_i = lax.broadcasted_iota(jnp.int32, (BLK, BLK), 1)
    tri = (row_i < col_i).astype(jnp.float32)
    excl = lax.dot_general(cnts, tri, (((1,), (0,)), ((), ())),
                           precision=HIGHEST)
    c0 = carry_ref[0]
    offs_i = (excl + c0.astype(jnp.float32)).astype(jnp.int32)
    offs_ref[...] = offs_i.reshape(1, 1, BLK)
    cnts_ref[...] = cnts.astype(jnp.int32).reshape(1, 1, BLK)
    esl_blk = esl_ref[0, pl.ds(pid * BLK, BLK)].reshape(1, BLK)
    denom_ref[...] = denom_ref[...] + jnp.sum(esl_blk * rowsum)
    rowlim_ref[...] = rowlim_ref[...] + jnp.sum(
        (offs_i < N).astype(jnp.int32)).reshape(1, 1)
    carry_ref[0] = c0 + jnp.sum(cnts).astype(jnp.int32)


def _passa(adj, x, W, b2, wl, wr, interpret=False):
    c = lambda i: (0, 0)
    return pl.pallas_call(
        _passa_body,
        grid=(NBLK,),
        in_specs=[
            pl.BlockSpec((BLK, N), lambda i: (i, 0)),
            pl.BlockSpec((N, D), c),
            pl.BlockSpec((D, D), c),
            pl.BlockSpec((1, D), c),
            pl.BlockSpec((1, D), c),
            pl.BlockSpec((1, D), c),
        ],
        out_specs=(
            pl.BlockSpec((1, 1, BLK), lambda i: (i, 0, 0)),
            pl.BlockSpec((1, 1, BLK), lambda i: (i, 0, 0)),
            pl.BlockSpec((1, 1), c),
            pl.BlockSpec((1, 1), c),
            pl.BlockSpec((BLK, N), lambda i: (i, 0)),
            pl.BlockSpec((N, D), c),
            pl.BlockSpec((1, N), c),
            pl.BlockSpec((1, N), c),
        ),
        out_shape=(
            jax.ShapeDtypeStruct((NBLK, 1, BLK), jnp.int32),
            jax.ShapeDtypeStruct((NBLK, 1, BLK), jnp.int32),
            jax.ShapeDtypeStruct((1, 1), jnp.float32),
            jax.ShapeDtypeStruct((1, 1), jnp.int32),
            jax.ShapeDtypeStruct((N, N), jnp.int8),
            jax.ShapeDtypeStruct((N, D), jnp.float32),
            jax.ShapeDtypeStruct((1, N), jnp.float32),
            jax.ShapeDtypeStruct((1, N), jnp.float32),
        ),
        scratch_shapes=[pltpu.SMEM((1,), jnp.int32)],
        interpret=interpret,
    )(adj, x, W, b2, wl, wr)


def _extract_sc(adj, offs, cnts, rowlim, esl1, esr1):
    mesh = plsc.VectorSubcoreMesh(core_axis_name="c", subcore_axis_name="s")
    cp = pltpu.CompilerParams()
    if "needs_layout_passes" in pltpu.CompilerParams.__dataclass_fields__:
        cp = dataclasses.replace(cp, needs_layout_passes=False)

    @functools.partial(
        pl.kernel,
        out_type=jax.ShapeDtypeStruct((32, N), jnp.float32),
        mesh=mesh,
        compiler_params=cp,
        scratch_types=[
            pltpu.VMEM((N,), jnp.int32),
            pltpu.VMEM((N,), jnp.int32),
            pltpu.VMEM((N,), jnp.int32),
            pltpu.VMEM((16,), jnp.int32),
            pltpu.VMEM((N,), jnp.float32),
            pltpu.VMEM((N,), jnp.float32),
            pltpu.VMEM((N,), jnp.float32),
        ],
    )
    def k(adj_hbm, offs_hbm, cnts_hbm, rowlim_hbm, esl_hbm, esr_hbm, out_hbm,
          row_v, offs_v, cnts_v, rl_v, esl_v, esr_v, acc_v):
        wid = lax.axis_index("s") * 2 + lax.axis_index("c")
        pltpu.sync_copy(rowlim_hbm, rl_v)
        rl = jnp.max(plsc.load_gather(rl_v, [jnp.zeros((16,), jnp.int32)]))
        pltpu.sync_copy(offs_hbm.at[pl.ds(0, 512)], offs_v.at[pl.ds(0, 512)])
        pltpu.sync_copy(cnts_hbm.at[pl.ds(0, 512)], cnts_v.at[pl.ds(0, 512)])
        pltpu.sync_copy(esl_hbm.at[pl.ds(0, 512)], esl_v.at[pl.ds(0, 512)])

        @pl.when(rl > 512)
        def _():
            pltpu.sync_copy(offs_hbm.at[pl.ds(512, N - 512)],
                            offs_v.at[pl.ds(512, N - 512)])
            pltpu.sync_copy(cnts_hbm.at[pl.ds(512, N - 512)],
                            cnts_v.at[pl.ds(512, N - 512)])
            pltpu.sync_copy(esl_hbm.at[pl.ds(512, N - 512)],
                            esl_v.at[pl.ds(512, N - 512)])

        pltpu.sync_copy(esr_hbm, esr_v)

        @pl.loop(0, N, step=16)
        def _(c):
            acc_v[pl.ds(c, 16)] = jnp.zeros((16,), jnp.float32)

        nt = jnp.maximum(rl - wid + 31, 0) // 32

        def trip(t, _):
            i = t * 32 + wid
            iv = jnp.full((16,), i, jnp.int32)
            off = jnp.max(plsc.load_gather(offs_v, [iv]))
            cnt = jnp.max(plsc.load_gather(cnts_v, [iv]))

            @pl.when((off < N) & (cnt > 0))
            def _():
                pltpu.sync_copy(adj_hbm.at[i], row_v)
                eslb = plsc.load_gather(esl_v, [iv])

                def chunk(c, tot):
                    v = row_v[pl.ds(c * 16, 16)]
                    m = v == 1
                    ones = jnp.where(m, 1, 0).astype(jnp.int32)
                    cs = plsc.cumsum(ones)
                    rank = off + tot + cs - 1
                    valid = m & (rank < N)
                    rank_safe = jnp.where(valid, rank, 0)
                    val = eslb * esr_v[pl.ds(c * 16, 16)]
                    plsc.store_scatter(acc_v, [rank_safe], val, mask=valid)
                    return tot + jnp.sum(ones)

                lax.fori_loop(0, N // 16, chunk, jnp.int32(0))

            return 0

        lax.fori_loop(0, nt, trip, jnp.int32(0))

        pltpu.sync_copy(acc_v, out_hbm.at[wid])

    return k(adj, offs, cnts, rowlim, esl1, esr1)


def _passb_body(mask8_ref, aparts_ref, xl_ref, denom_ref, out_ref):
    mask = mask8_ref[...].astype(jnp.float32)
    a = jnp.sum(aparts_ref[...], axis=0, keepdims=True)
    d = denom_ref[...]
    d = jnp.where(d > 0.0, d, 1.0)
    mw = mask * (a / d)
    ob = lax.dot_general(mw, xl_ref[...], (((1,), (0,)), ((), ())))
    out_ref[...] = jnp.maximum(ob, 0.0)


def _passb(mask8, aparts, xl, denom, interpret=False):
    return pl.pallas_call(
        _passb_body,
        grid=(NBLK,),
        in_specs=[
            pl.BlockSpec((BLK, N), lambda i: (i, 0)),
            pl.BlockSpec((32, N), lambda i: (0, 0)),
            pl.BlockSpec((N, D), lambda i: (0, 0)),
            pl.BlockSpec((1, 1), lambda i: (0, 0)),
        ],
        out_specs=pl.BlockSpec((BLK, D), lambda i: (i, 0)),
        out_shape=jax.ShapeDtypeStruct((N, D), jnp.float32),
        interpret=interpret,
    )(mask8, aparts, xl, denom)


def kernel(x, adj, W, b, att_w, att_b):
    wl = att_w[:, :D]
    wr = att_w[:, D:]
    b2 = b.reshape(1, D)
    offs3, cnts3, denom, rowlim, mask8, xl, esl, esr = _passa(
        adj, x, W, b2, wl, wr)
    offs = offs3.reshape(N)
    cnts = cnts3.reshape(N)
    rl16 = jnp.broadcast_to(rowlim.reshape(1), (16,))
    aparts = _extract_sc(adj, offs, cnts, rl16, esl.reshape(N),
                         esr.reshape(N))
    return _passb(mask8, aparts, xl, denom)


# speedup vs baseline: 256.0608x; 1.1005x over previous
import dataclasses
import functools

import jax
import jax.numpy as jnp
from jax import lax
from jax.experimental import pallas as pl
from jax.experimental.pallas import tpu as pltpu
from jax.experimental.pallas import tpu_sc as plsc

N = 4096
D = 128
BLK = 512
NBLK = N // BLK
HIGHEST = lax.Precision.HIGHEST


def _passa_body(adj_ref, x_ref, w_ref, b_ref, wl_ref, wr_ref,
                offs_ref, cnts_ref, denom_ref, rowlim_ref, mask8_ref,
                xl_ref, esl_ref, esr_ref,
                carry_ref):
    pid = pl.program_id(0)

    @pl.when(pid == 0)
    def _():
        carry_ref[0] = 0
        denom_ref[...] = jnp.zeros((1, 1), jnp.float32)
        rowlim_ref[...] = jnp.zeros((1, 1), jnp.int32)
        xl = lax.dot_general(x_ref[...], w_ref[...], (((1,), (1,)), ((), ())),
                             precision=HIGHEST) + b_ref[...]
        xl_ref[...] = xl
        sl = lax.dot_general(wl_ref[...], xl, (((1,), (1,)), ((), ())),
                             precision=HIGHEST)
        sr = lax.dot_general(wr_ref[...], xl, (((1,), (1,)), ((), ())),
                             precision=HIGHEST)
        esl_ref[...] = jnp.exp(sl - jnp.max(sl))
        esr_ref[...] = jnp.exp(sr - jnp.max(sr))

    mask = (adj_ref[...] == 1).astype(jnp.float32)
    mask8_ref[...] = mask.astype(jnp.int8)
    lhs = jnp.concatenate([esr_ref[...], jnp.ones((1, N), jnp.float32)], 0)
    rc = lax.dot_general(lhs, mask, (((1,), (1,)), ((), ())))
    rowsum = rc[0:1, :]
    cnts = rc[1:2, :]
    row_i = lax.broadcasted_iota(jnp.int32, (BLK, BLK), 0)
    col_i = lax.broadcasted_iota(jnp.int32, (BLK, BLK), 1)
    tri = (row_i < col_i).astype(jnp.float32)
    excl = lax.dot_general(cnts, tri, (((1,), (0,)), ((), ())),
                           precision=HIGHEST)
    c0 = carry_ref[0]
    offs_i = (excl + c0.astype(jnp.float32)).astype(jnp.int32)
    offs_ref[...] = offs_i.reshape(1, 1, BLK)
    cnts_ref[...] = cnts.astype(jnp.int32).reshape(1, 1, BLK)
    esl_blk = esl_ref[0, pl.ds(pid * BLK, BLK)].reshape(1, BLK)
    denom_ref[...] = denom_ref[...] + jnp.sum(esl_blk * rowsum)
    rowlim_ref[...] = rowlim_ref[...] + jnp.sum(
        (offs_i < N).astype(jnp.int32)).reshape(1, 1)
    carry_ref[0] = c0 + jnp.sum(cnts).astype(jnp.int32)


def _passa(adj, x, W, b2, wl, wr, interpret=False):
    c = lambda i: (0, 0)
    return pl.pallas_call(
        _passa_body,
        grid=(NBLK,),
        in_specs=[
            pl.BlockSpec((BLK, N), lambda i: (i, 0)),
            pl.BlockSpec((N, D), c),
            pl.BlockSpec((D, D), c),
            pl.BlockSpec((1, D), c),
            pl.BlockSpec((1, D), c),
            pl.BlockSpec((1, D), c),
        ],
        out_specs=(
            pl.BlockSpec((1, 1, BLK), lambda i: (i, 0, 0)),
            pl.BlockSpec((1, 1, BLK), lambda i: (i, 0, 0)),
            pl.BlockSpec((1, 1), c),
            pl.BlockSpec((1, 1), c),
            pl.BlockSpec((BLK, N), lambda i: (i, 0)),
            pl.BlockSpec((N, D), c),
            pl.BlockSpec((1, N), c),
            pl.BlockSpec((1, N), c),
        ),
        out_shape=(
            jax.ShapeDtypeStruct((NBLK, 1, BLK), jnp.int32),
            jax.ShapeDtypeStruct((NBLK, 1, BLK), jnp.int32),
            jax.ShapeDtypeStruct((1, 1), jnp.float32),
            jax.ShapeDtypeStruct((1, 1), jnp.int32),
            jax.ShapeDtypeStruct((N, N), jnp.int8),
            jax.ShapeDtypeStruct((N, D), jnp.float32),
            jax.ShapeDtypeStruct((1, N), jnp.float32),
            jax.ShapeDtypeStruct((1, N), jnp.float32),
        ),
        scratch_shapes=[pltpu.SMEM((1,), jnp.int32)],
        interpret=interpret,
    )(adj, x, W, b2, wl, wr)


def _extract_sc(adj, offs, cnts, rowlim, esl1, esr1):
    mesh = plsc.VectorSubcoreMesh(core_axis_name="c", subcore_axis_name="s")
    cp = pltpu.CompilerParams()
    if "needs_layout_passes" in pltpu.CompilerParams.__dataclass_fields__:
        cp = dataclasses.replace(cp, needs_layout_passes=False)

    @functools.partial(
        pl.kernel,
        out_type=jax.ShapeDtypeStruct((32, N), jnp.float32),
        mesh=mesh,
        compiler_params=cp,
        scratch_types=[
            pltpu.VMEM((N,), jnp.int32),
            pltpu.VMEM((N,), jnp.int32),
            pltpu.VMEM((N,), jnp.int32),
            pltpu.VMEM((16,), jnp.int32),
            pltpu.VMEM((N,), jnp.float32),
            pltpu.VMEM((N,), jnp.float32),
            pltpu.VMEM((N,), jnp.float32),
        ],
    )
    def k(adj_hbm, offs_hbm, cnts_hbm, rowlim_hbm, esl_hbm, esr_hbm, out_hbm,
          row_v, offs_v, cnts_v, rl_v, esl_v, esr_v, acc_v):
        wid = lax.axis_index("s") * 2 + lax.axis_index("c")
        pltpu.sync_copy(rowlim_hbm, rl_v)
        rl = jnp.max(plsc.load_gather(rl_v, [jnp.zeros((16,), jnp.int32)]))
        pltpu.sync_copy(offs_hbm.at[pl.ds(0, 512)], offs_v.at[pl.ds(0, 512)])
        pltpu.sync_copy(cnts_hbm.at[pl.ds(0, 512)], cnts_v.at[pl.ds(0, 512)])
        pltpu.sync_copy(esl_hbm.at[pl.ds(0, 512)], esl_v.at[pl.ds(0, 512)])

        @pl.when(rl > 512)
        def _():
            pltpu.sync_copy(offs_hbm.at[pl.ds(512, N - 512)],
                            offs_v.at[pl.ds(512, N - 512)])
            pltpu.sync_copy(cnts_hbm.at[pl.ds(512, N - 512)],
                            cnts_v.at[pl.ds(512, N - 512)])
            pltpu.sync_copy(esl_hbm.at[pl.ds(512, N - 512)],
                            esl_v.at[pl.ds(512, N - 512)])

        pltpu.sync_copy(esr_hbm, esr_v)

        @pl.loop(0, N, step=16)
        def _(c):
            acc_v[pl.ds(c, 16)] = jnp.zeros((16,), jnp.float32)

        nt = jnp.maximum(rl - wid + 31, 0) // 32

        def trip(t, _):
            i = t * 32 + wid
            iv = jnp.full((16,), i, jnp.int32)
            off = jnp.max(plsc.load_gather(offs_v, [iv]))
            cnt = jnp.max(plsc.load_gather(cnts_v, [iv]))

            @pl.when((off < N) & (cnt > 0))
            def _():
                pltpu.sync_copy(adj_hbm.at[i], row_v)
                eslb = plsc.load_gather(esl_v, [iv])

                def chunk(c, tot):
                    v = row_v[pl.ds(c * 16, 16)]
                    m = v == 1
                    ones = jnp.where(m, 1, 0).astype(jnp.int32)
                    cs = plsc.cumsum(ones)
                    rank = off + tot + cs - 1
                    valid = m & (rank < N)
                    rank_safe = jnp.where(valid, rank, 0)
                    val = eslb * esr_v[pl.ds(c * 16, 16)]
                    plsc.store_scatter(acc_v, [rank_safe], val, mask=valid)
                    return tot + jnp.sum(ones)

                lax.fori_loop(0, N // 16, chunk, jnp.int32(0))

            return 0

        lax.fori_loop(0, nt, trip, jnp.int32(0))

        pltpu.sync_copy(acc_v, out_hbm.at[wid])

    return k(adj, offs, cnts, rowlim, esl1, esr1)


def _passb_body(mask8_ref, aparts_ref, xl_ref, denom_ref, out_ref):
    mask = mask8_ref[...].astype(jnp.float32)
    a = jnp.sum(aparts_ref[...], axis=0, keepdims=True)
    d = denom_ref[...]
    d = jnp.where(d > 0.0, d, 1.0)
    mw = mask * (a / d)
    ob = lax.dot_general(mw, xl_ref[...], (((1,), (0,)), ((), ())))
    out_ref[...] = jnp.maximum(ob, 0.0)


def _passb(mask8, aparts, xl, denom, interpret=False):
    return pl.pallas_call(
        _passb_body,
        grid=(NBLK,),
        in_specs=[
            pl.BlockSpec((BLK, N), lambda i: (i, 0)),
            pl.BlockSpec((32, N), lambda i: (0, 0)),
            pl.BlockSpec((N, D), lambda i: (0, 0)),
            pl.BlockSpec((1, 1), lambda i: (0, 0)),
        ],
        out_specs=pl.BlockSpec((BLK, D), lambda i: (i, 0)),
        out_shape=jax.ShapeDtypeStruct((N, D), jnp.float32),
        interpret=interpret,
    )(mask8, aparts, xl, denom)


def kernel(x, adj, W, b, att_w, att_b):
    wl = att_w[:, :D]
    wr = att_w[:, D:]
    b2 = b.reshape(1, D)
    offs3, cnts3, denom, rowlim, mask8, xl, esl, esr = _passa(
        adj, x, W, b2, wl, wr)
    offs = offs3.reshape(N)
    cnts = cnts3.reshape(N)
    rl16 = jnp.broadcast_to(rowlim.reshape(1), (16,))
    aparts = _extract_sc(adj, offs, cnts, rl16, esl.reshape(N),
                         esr.reshape(N))
    return _passb(mask8, aparts, xl, denom)


# speedup vs baseline: 260.6236x; 1.0178x over previous
import dataclasses
import functools

import jax
import jax.numpy as jnp
from jax import lax
from jax.experimental import pallas as pl
from jax.experimental.pallas import tpu as pltpu
from jax.experimental.pallas import tpu_sc as plsc

N = 4096
D = 128
BLK = 1024
NBLK = N // BLK
HIGHEST = lax.Precision.HIGHEST


def _passa_body(adj_ref, x_ref, w_ref, b_ref, wl_ref, wr_ref,
                offs_ref, cnts_ref, denom_ref, rowlim_ref, mask8_ref,
                xl_ref, esl_ref, esr_ref,
                carry_ref):
    pid = pl.program_id(0)

    @pl.when(pid == 0)
    def _():
        carry_ref[0] = 0
        denom_ref[...] = jnp.zeros((1, 1), jnp.float32)
        rowlim_ref[...] = jnp.zeros((1, 1), jnp.int32)
        xl = lax.dot_general(x_ref[...], w_ref[...], (((1,), (1,)), ((), ())),
                             precision=HIGHEST) + b_ref[...]
        xl_ref[...] = xl
        sl = lax.dot_general(wl_ref[...], xl, (((1,), (1,)), ((), ())),
                             precision=HIGHEST)
        sr = lax.dot_general(wr_ref[...], xl, (((1,), (1,)), ((), ())),
                             precision=HIGHEST)
        esl_ref[...] = jnp.exp(sl - jnp.max(sl))
        esr_ref[...] = jnp.exp(sr - jnp.max(sr))

    mask = (adj_ref[...] == 1).astype(jnp.float32)
    mask8_ref[...] = mask.astype(jnp.int8)
    lhs = jnp.concatenate([esr_ref[...], jnp.ones((1, N), jnp.float32)], 0)
    rc = lax.dot_general(lhs, mask, (((1,), (1,)), ((), ())))
    rowsum = rc[0:1, :]
    cnts = rc[1:2, :]
    row_i = lax.broadcasted_iota(jnp.int32, (BLK, BLK), 0)
    col_i = lax.broadcasted_iota(jnp.int32, (BLK, BLK), 1)
    tri = (row_i < col_i).astype(jnp.float32)
    excl = lax.dot_general(cnts, tri, (((1,), (0,)), ((), ())),
                           precision=HIGHEST)
    c0 = carry_ref[0]
    offs_i = (excl + c0.astype(jnp.float32)).astype(jnp.int32)
    offs_ref[...] = offs_i.reshape(1, 1, BLK)
    cnts_ref[...] = cnts.astype(jnp.int32).reshape(1, 1, BLK)
    esl_blk = esl_ref[0, pl.ds(pid * BLK, BLK)].reshape(1, BLK)
    denom_ref[...] = denom_ref[...] + jnp.sum(esl_blk * rowsum)
    rowlim_ref[...] = rowlim_ref[...] + jnp.sum(
        (offs_i < N).astype(jnp.int32)).reshape(1, 1)
    carry_ref[0] = c0 + jnp.sum(cnts).astype(jnp.int32)


def _passa(adj, x, W, b2, wl, wr, interpret=False):
    c = lambda i: (0, 0)
    return pl.pallas_call(
        _passa_body,
        grid=(NBLK,),
        in_specs=[
            pl.BlockSpec((BLK, N), lambda i: (i, 0)),
            pl.BlockSpec((N, D), c),
            pl.BlockSpec((D, D), c),
            pl.BlockSpec((1, D), c),
            pl.BlockSpec((1, D), c),
            pl.BlockSpec((1, D), c),
        ],
        out_specs=(
            pl.BlockSpec((1, 1, BLK), lambda i: (i, 0, 0)),
            pl.BlockSpec((1, 1, BLK), lambda i: (i, 0, 0)),
            pl.BlockSpec((1, 1), c),
            pl.BlockSpec((1, 1), c),
            pl.BlockSpec((BLK, N), lambda i: (i, 0)),
            pl.BlockSpec((N, D), c),
            pl.BlockSpec((1, N), c),
            pl.BlockSpec((1, N), c),
        ),
        out_shape=(
            jax.ShapeDtypeStruct((NBLK, 1, BLK), jnp.int32),
            jax.ShapeDtypeStruct((NBLK, 1, BLK), jnp.int32),
            jax.ShapeDtypeStruct((1, 1), jnp.float32),
            jax.ShapeDtypeStruct((1, 1), jnp.int32),
            jax.ShapeDtypeStruct((N, N), jnp.int8),
            jax.ShapeDtypeStruct((N, D), jnp.float32),
            jax.ShapeDtypeStruct((1, N), jnp.float32),
            jax.ShapeDtypeStruct((1, N), jnp.float32),
        ),
        scratch_shapes=[pltpu.SMEM((1,), jnp.int32)],
        interpret=interpret,
    )(adj, x, W, b2, wl, wr)


def _extract_sc(adj, offs, cnts, rowlim, esl1, esr1):
    mesh = plsc.VectorSubcoreMesh(core_axis_name="c", subcore_axis_name="s")
    cp = pltpu.CompilerParams()
    if "needs_layout_passes" in pltpu.CompilerParams.__dataclass_fields__:
        cp = dataclasses.replace(cp, needs_layout_passes=False)

    @functools.partial(
        pl.kernel,
        out_type=jax.ShapeDtypeStruct((32, N), jnp.float32),
        mesh=mesh,
        compiler_params=cp,
        scratch_types=[
            pltpu.VMEM((N,), jnp.int32),
            pltpu.VMEM((N,), jnp.int32),
            pltpu.VMEM((N,), jnp.int32),
            pltpu.VMEM((16,), jnp.int32),
            pltpu.VMEM((N,), jnp.float32),
            pltpu.VMEM((N,), jnp.float32),
            pltpu.VMEM((N,), jnp.float32),
        ],
    )
    def k(adj_hbm, offs_hbm, cnts_hbm, rowlim_hbm, esl_hbm, esr_hbm, out_hbm,
          row_v, offs_v, cnts_v, rl_v, esl_v, esr_v, acc_v):
        wid = lax.axis_index("s") * 2 + lax.axis_index("c")
        pltpu.sync_copy(rowlim_hbm, rl_v)
        rl = jnp.max(plsc.load_gather(rl_v, [jnp.zeros((16,), jnp.int32)]))
        pltpu.sync_copy(offs_hbm.at[pl.ds(0, 512)], offs_v.at[pl.ds(0, 512)])
        pltpu.sync_copy(cnts_hbm.at[pl.ds(0, 512)], cnts_v.at[pl.ds(0, 512)])
        pltpu.sync_copy(esl_hbm.at[pl.ds(0, 512)], esl_v.at[pl.ds(0, 512)])

        @pl.when(rl > 512)
        def _():
            pltpu.sync_copy(offs_hbm.at[pl.ds(512, N - 512)],
                            offs_v.at[pl.ds(512, N - 512)])
            pltpu.sync_copy(cnts_hbm.at[pl.ds(512, N - 512)],
                            cnts_v.at[pl.ds(512, N - 512)])
            pltpu.sync_copy(esl_hbm.at[pl.ds(512, N - 512)],
                            esl_v.at[pl.ds(512, N - 512)])

        pltpu.sync_copy(esr_hbm, esr_v)

        @pl.loop(0, N, step=16)
        def _(c):
            acc_v[pl.ds(c, 16)] = jnp.zeros((16,), jnp.float32)

        nt = jnp.maximum(rl - wid + 31, 0) // 32

        def trip(t, _):
            i = t * 32 + wid
            iv = jnp.full((16,), i, jnp.int32)
            off = jnp.max(plsc.load_gather(offs_v, [iv]))
            cnt = jnp.max(plsc.load_gather(cnts_v, [iv]))

            @pl.when((off < N) & (cnt > 0))
            def _():
                pltpu.sync_copy(adj_hbm.at[i], row_v)
                eslb = plsc.load_gather(esl_v, [iv])

                def chunk(c, tot):
                    v = row_v[pl.ds(c * 16, 16)]
                    m = v == 1
                    ones = jnp.where(m, 1, 0).astype(jnp.int32)
                    cs = plsc.cumsum(ones)
                    rank = off + tot + cs - 1
                    valid = m & (rank < N)
                    rank_safe = jnp.where(valid, rank, 0)
                    val = eslb * esr_v[pl.ds(c * 16, 16)]
                    plsc.store_scatter(acc_v, [rank_safe], val, mask=valid)
                    return tot + jnp.sum(ones)

                lax.fori_loop(0, N // 16, chunk, jnp.int32(0))

            return 0

        lax.fori_loop(0, nt, trip, jnp.int32(0))

        pltpu.sync_copy(acc_v, out_hbm.at[wid])

    return k(adj, offs, cnts, rowlim, esl1, esr1)


def _passb_body(mask8_ref, aparts_ref, xl_ref, denom_ref, out_ref):
    mask = mask8_ref[...].astype(jnp.float32)
    a = jnp.sum(aparts_ref[...], axis=0, keepdims=True)
    d = denom_ref[...]
    d = jnp.where(d > 0.0, d, 1.0)
    mw = mask * (a / d)
    ob = lax.dot_general(mw, xl_ref[...], (((1,), (0,)), ((), ())))
    out_ref[...] = jnp.maximum(ob, 0.0)


def _passb(mask8, aparts, xl, denom, interpret=False):
    return pl.pallas_call(
        _passb_body,
        grid=(NBLK,),
        in_specs=[
            pl.BlockSpec((BLK, N), lambda i: (i, 0)),
            pl.BlockSpec((32, N), lambda i: (0, 0)),
            pl.BlockSpec((N, D), lambda i: (0, 0)),
            pl.BlockSpec((1, 1), lambda i: (0, 0)),
        ],
        out_specs=pl.BlockSpec((BLK, D), lambda i: (i, 0)),
        out_shape=jax.ShapeDtypeStruct((N, D), jnp.float32),
        interpret=interpret,
    )(mask8, aparts, xl, denom)


def kernel(x, adj, W, b, att_w, att_b):
    wl = att_w[:, :D]
    wr = att_w[:, D:]
    b2 = b.reshape(1, D)
    offs3, cnts3, denom, rowlim, mask8, xl, esl, esr = _passa(
        adj, x, W, b2, wl, wr)
    offs = offs3.reshape(N)
    cnts = cnts3.reshape(N)
    rl16 = jnp.broadcast_to(rowlim.reshape(1), (16,))
    aparts = _extract_sc(adj, offs, cnts, rl16, esl.reshape(N),
                         esr.reshape(N))
    return _passb(mask8, aparts, xl, denom)
